# Initial kernel scaffold; baseline (speedup 1.0000x reference)
import jax
import jax.numpy as jnp
from jax.experimental import pallas as pl


def kernel(v):
    raise NotImplementedError("write your pallas kernel here")


# speedup vs baseline: 11.2737x; 11.2737x over previous
import functools
import numpy as np
import jax
import jax.numpy as jnp
from jax import lax
from jax.experimental import pallas as pl
from jax.experimental.pallas import tpu as pltpu
from jax.experimental.pallas import tpu_sc as plsc

_LAYER_IDX = 6
_NUM_LAYERS = 12
_DIM = 2048

_NW = 32
_R = 512
_MAXOFF = 128


def _shift_offsets(v_len, d_v):
    i = np.arange(d_v, dtype=np.float64)
    e = (_LAYER_IDX * _DIM + i) / (_NUM_LAYERS * _DIM)
    return np.ceil(np.power(float(v_len), e)).astype(np.int64)


@functools.lru_cache(maxsize=None)
def _build(B, N, D):
    off = _shift_offsets(N, D)
    assert off.min() >= 1 and off.max() <= _MAXOFF
    delta_np = (_MAXOFF - off).astype(np.int32)
    span = int(_MAXOFF - off.min())
    CPW = D // _NW
    S = _R + span
    NBLK = N // _R
    assert N % _R == 0 and D % _NW == 0 and CPW % 16 == 0

    mesh = plsc.VectorSubcoreMesh(core_axis_name="c", subcore_axis_name="s")

    @functools.partial(
        pl.kernel,
        out_type=jax.ShapeDtypeStruct((B, N, D), jnp.float32),
        mesh=mesh,
        compiler_params=pltpu.CompilerParams(
            use_tc_tiling_on_sc=False, needs_layout_passes=False),
        scratch_types=[
            pltpu.VMEM((CPW,), jnp.int32),
            pltpu.VMEM((S, CPW), jnp.float32),
            pltpu.VMEM((_R, CPW), jnp.float32),
        ],
    )
    def swd(v_hbm, delta_hbm, out_hbm, delta_v, src_v, dst_v):
        wid = lax.axis_index("s") * 2 + lax.axis_index("c")
        c0 = wid * CPW
        pltpu.sync_copy(delta_hbm.at[pl.ds(c0, CPW)], delta_v)
        iota = lax.iota(jnp.int32, 16)
        deltas = [delta_v[pl.ds(c * 16, 16)] for c in range(CPW // 16)]
        cols = [c * 16 + iota for c in range(CPW // 16)]

        def run_block(b, jblk):
            j0 = jblk * _R
            if jblk == 0:
                pltpu.sync_copy(
                    v_hbm.at[b, pl.ds(N - _MAXOFF, _MAXOFF), pl.ds(c0, CPW)],
                    src_v.at[pl.ds(0, _MAXOFF)])
                pltpu.sync_copy(
                    v_hbm.at[b, pl.ds(0, S - _MAXOFF), pl.ds(c0, CPW)],
                    src_v.at[pl.ds(_MAXOFF, S - _MAXOFF)])
            else:
                pltpu.sync_copy(
                    v_hbm.at[b, pl.ds(j0 - _MAXOFF, S), pl.ds(c0, CPW)],
                    src_v)

            def pair(p, carry):
                r = 2 * p
                for c in range(CPW // 16):
                    ridx = deltas[c] + r
                    lo = plsc.load_gather(src_v, [ridx, cols[c]])
                    hi = plsc.load_gather(src_v, [ridx + 1, cols[c]])
                    plsc.store_scatter(dst_v, [jnp.zeros((16,), jnp.int32) + r, cols[c]],
                                       jnp.minimum(lo, hi))
                    plsc.store_scatter(dst_v, [jnp.zeros((16,), jnp.int32) + (r + 1), cols[c]],
                                       jnp.maximum(lo, hi))
                return carry

            lax.fori_loop(0, _R // 2, pair, 0)
            pltpu.sync_copy(dst_v, out_hbm.at[b, pl.ds(j0, _R), pl.ds(c0, CPW)])

        def per_batch(b, carry):
            for jblk in range(NBLK):
                run_block(b, jblk)
            return carry

        lax.fori_loop(0, B, per_batch, 0)

    def call(v, delta):
        return swd(v, delta)

    return call, jnp.asarray(delta_np)


def kernel(v):
    B, N, D = v.shape
    call, delta = _build(B, N, D)
    return call(v, delta)


# speedup vs baseline: 14.4113x; 1.2783x over previous
import functools
import numpy as np
import jax
import jax.numpy as jnp
from jax import lax
from jax.experimental import pallas as pl
from jax.experimental.pallas import tpu as pltpu
from jax.experimental.pallas import tpu_sc as plsc

_LAYER_IDX = 6
_NUM_LAYERS = 12
_DIM = 2048

_NW = 32
_R = 256
_MAXOFF = 128
_NBUF = 2


def _shift_offsets(v_len, d_v):
    i = np.arange(d_v, dtype=np.float64)
    e = (_LAYER_IDX * _DIM + i) / (_NUM_LAYERS * _DIM)
    return np.ceil(np.power(float(v_len), e)).astype(np.int64)


@functools.lru_cache(maxsize=None)
def _build(B, N, D):
    off = _shift_offsets(N, D)
    assert off.min() >= 1 and off.max() <= _MAXOFF
    delta_np = (_MAXOFF - off).astype(np.int32)
    CPW = D // _NW
    NC16 = CPW // 16
    S = _R + _MAXOFF - int(off.min())
    NBLK = N // _R
    T = B * NBLK
    assert N % _R == 0 and D % _NW == 0 and CPW % 16 == 0 and T % _NBUF == 0

    mesh = plsc.VectorSubcoreMesh(core_axis_name="c", subcore_axis_name="s")

    @functools.partial(
        pl.kernel,
        out_type=jax.ShapeDtypeStruct((B, N, D), jnp.float32),
        mesh=mesh,
        compiler_params=pltpu.CompilerParams(
            use_tc_tiling_on_sc=False, needs_layout_passes=False),
        scratch_types=[
            pltpu.VMEM((CPW,), jnp.int32),
            pltpu.VMEM((_NBUF, S, CPW), jnp.float32),
            pltpu.VMEM((_NBUF, _R, CPW), jnp.float32),
            pltpu.SemaphoreType.DMA,
            pltpu.SemaphoreType.DMA,
            pltpu.SemaphoreType.DMA,
            pltpu.SemaphoreType.DMA,
        ],
    )
    def swd(v_hbm, delta_hbm, out_hbm, delta_v, src_v, dst_v,
            semi0, semi1, semo0, semo1):
        wid = lax.axis_index("s") * 2 + lax.axis_index("c")
        c0 = wid * CPW
        pltpu.sync_copy(delta_hbm.at[pl.ds(c0, CPW)], delta_v)
        iota = lax.iota(jnp.int32, 16)
        deltas = [delta_v[pl.ds(c * 16, 16)] for c in range(NC16)]
        cols = [c * 16 + iota for c in range(NC16)]
        semi = [semi0, semi1]
        semo = [semo0, semo1]

        def start_in(t, k, sem):
            b = t // NBLK
            jblk = t % NBLK
            j0 = jblk * _R
            sbuf = src_v.at[k]

            @pl.when(jblk == 0)
            def _():
                pltpu.async_copy(
                    v_hbm.at[b, pl.ds(N - _MAXOFF, _MAXOFF), pl.ds(c0, CPW)],
                    sbuf.at[pl.ds(0, _MAXOFF)], sem)
                pltpu.async_copy(
                    v_hbm.at[b, pl.ds(0, S - _MAXOFF), pl.ds(c0, CPW)],
                    sbuf.at[pl.ds(_MAXOFF, S - _MAXOFF)], sem)

            @pl.when(jblk != 0)
            def _():
                pltpu.async_copy(
                    v_hbm.at[b, pl.ds(j0 - _MAXOFF, S), pl.ds(c0, CPW)],
                    sbuf, sem)

        def wait_in(k, sem):
            pltpu.make_async_copy(
                v_hbm.at[0, pl.ds(0, S), pl.ds(c0, CPW)],
                src_v.at[k], sem).wait()

        def start_out(t, k, sem):
            b = t // NBLK
            j0 = (t % NBLK) * _R
            pltpu.async_copy(
                dst_v.at[k], out_hbm.at[b, pl.ds(j0, _R), pl.ds(c0, CPW)],
                sem)

        def wait_out(k, sem):
            pltpu.make_async_copy(
                dst_v.at[k], out_hbm.at[0, pl.ds(0, _R), pl.ds(c0, CPW)],
                sem).wait()

        def compute(k):
            sbuf = src_v.at[k]
            dbuf = dst_v.at[k]

            @pl.loop(0, _R // 2, unroll=2)
            def pair(p):
                r = 2 * p
                for c in range(NC16):
                    ridx = deltas[c] + r
                    lo = plsc.load_gather(sbuf, [ridx, cols[c]])
                    hi = plsc.load_gather(sbuf, [ridx + 1, cols[c]])
                    dbuf[r, pl.ds(c * 16, 16)] = jnp.minimum(lo, hi)
                    dbuf[r + 1, pl.ds(c * 16, 16)] = jnp.maximum(lo, hi)

        start_in(0, 0, semi[0])

        @pl.loop(0, T, step=_NBUF)
        def task_loop(t):
            for k in range(_NBUF):
                tk = t + k

                @pl.when(tk + 1 < T)
                def _():
                    start_in(tk + 1, (k + 1) % _NBUF, semi[(k + 1) % _NBUF])

                wait_in(k, semi[k])

                @pl.when(tk >= _NBUF)
                def _():
                    wait_out(k, semo[k])

                compute(k)
                start_out(tk, k, semo[k])

        for k in range(_NBUF):
            wait_out(k, semo[k])

    def call(v, delta):
        return swd(v, delta)

    return call, jnp.asarray(delta_np)


def kernel(v):
    B, N, D = v.shape
    call, delta = _build(B, N, D)
    return call(v, delta)


# speedup vs baseline: 29.7094x; 2.0615x over previous
import functools
import numpy as np
import jax
import jax.numpy as jnp
from jax import lax
from jax.experimental import pallas as pl
from jax.experimental.pallas import tpu as pltpu
from jax.experimental.pallas import tpu_sc as plsc

_LAYER_IDX = 6
_NUM_LAYERS = 12
_DIM = 2048

_NW = 32
_NSTRIPE = 16
_R = 128
_MAXOFF = 128
_NBUF = 2


def _shift_offsets(v_len, d_v):
    i = np.arange(d_v, dtype=np.float64)
    e = (_LAYER_IDX * _DIM + i) / (_NUM_LAYERS * _DIM)
    return np.ceil(np.power(float(v_len), e)).astype(np.int64)


@functools.lru_cache(maxsize=None)
def _build(B, N, D):
    off = _shift_offsets(N, D)
    assert off.min() >= 1 and off.max() <= _MAXOFF
    delta_np = (_MAXOFF - off).astype(np.int32)
    CPW = D // _NSTRIPE
    NC16 = CPW // 16
    S = _R + _MAXOFF - int(off.min())
    NBLK = N // _R
    T = (B * NBLK) // 2
    assert N % _R == 0 and D % _NSTRIPE == 0 and CPW % 16 == 0
    assert T % _NBUF == 0

    mesh = plsc.VectorSubcoreMesh(core_axis_name="c", subcore_axis_name="s")

    @functools.partial(
        pl.kernel,
        out_type=jax.ShapeDtypeStruct((B, N, D), jnp.float32),
        mesh=mesh,
        compiler_params=pltpu.CompilerParams(needs_layout_passes=False),
        scratch_types=[
            pltpu.VMEM((CPW,), jnp.int32),
            pltpu.VMEM((_NBUF, S, CPW), jnp.float32),
            pltpu.VMEM((_NBUF, _R, CPW), jnp.float32),
            pltpu.SemaphoreType.DMA,
            pltpu.SemaphoreType.DMA,
            pltpu.SemaphoreType.DMA,
            pltpu.SemaphoreType.DMA,
        ],
    )
    def swd(v_hbm, delta_hbm, out_hbm, delta_v, src_v, dst_v,
            semi0, semi1, semo0, semo1):
        wid = lax.axis_index("s") * 2 + lax.axis_index("c")
        stripe = wid // 2
        half = wid % 2
        c0 = stripe * CPW
        t_base = half * T
        pltpu.sync_copy(delta_hbm.at[pl.ds(c0, CPW)], delta_v)
        iota = lax.iota(jnp.int32, 16)
        deltas = [delta_v[pl.ds(c * 16, 16)] for c in range(NC16)]
        cols = [c * 16 + iota for c in range(NC16)]
        semi = [semi0, semi1]
        semo = [semo0, semo1]

        def start_in(t, k, sem):
            tt = t_base + t
            b = tt // NBLK
            jblk = tt % NBLK
            j0 = jblk * _R
            sbuf = src_v.at[k]

            @pl.when(jblk == 0)
            def _():
                pltpu.async_copy(
                    v_hbm.at[b, pl.ds(N - _MAXOFF, _MAXOFF), pl.ds(c0, CPW)],
                    sbuf.at[pl.ds(0, _MAXOFF)], sem)
                pltpu.async_copy(
                    v_hbm.at[b, pl.ds(0, S - _MAXOFF), pl.ds(c0, CPW)],
                    sbuf.at[pl.ds(_MAXOFF, S - _MAXOFF)], sem)

            @pl.when(jblk != 0)
            def _():
                pltpu.async_copy(
                    v_hbm.at[b, pl.ds(j0 - _MAXOFF, S), pl.ds(c0, CPW)],
                    sbuf, sem)

        def wait_in(k, sem):
            pltpu.make_async_copy(
                v_hbm.at[0, pl.ds(0, S), pl.ds(c0, CPW)],
                src_v.at[k], sem).wait()

        def start_out(t, k, sem):
            tt = t_base + t
            b = tt // NBLK
            j0 = (tt % NBLK) * _R
            pltpu.async_copy(
                dst_v.at[k], out_hbm.at[b, pl.ds(j0, _R), pl.ds(c0, CPW)],
                sem)

        def wait_out(k, sem):
            pltpu.make_async_copy(
                dst_v.at[k], out_hbm.at[0, pl.ds(0, _R), pl.ds(c0, CPW)],
                sem).wait()

        def compute(k):
            sbuf = src_v.at[k]
            dbuf = dst_v.at[k]

            @pl.loop(0, _R // 2, unroll=2)
            def pair(p):
                r = 2 * p
                for c in range(NC16):
                    ridx = deltas[c] + r
                    lo = plsc.load_gather(sbuf, [ridx, cols[c]])
                    hi = plsc.load_gather(sbuf, [ridx + 1, cols[c]])
                    dbuf[r, pl.ds(c * 16, 16)] = jnp.minimum(lo, hi)
                    dbuf[r + 1, pl.ds(c * 16, 16)] = jnp.maximum(lo, hi)

        start_in(0, 0, semi[0])

        @pl.loop(0, T, step=_NBUF)
        def task_loop(t):
            for k in range(_NBUF):
                tk = t + k

                @pl.when(tk + 1 < T)
                def _():
                    start_in(tk + 1, (k + 1) % _NBUF, semi[(k + 1) % _NBUF])

                wait_in(k, semi[k])

                @pl.when(tk >= _NBUF)
                def _():
                    wait_out(k, semo[k])

                compute(k)
                start_out(tk, k, semo[k])

        for k in range(_NBUF):
            wait_out(k, semo[k])

    def call(v, delta):
        return swd(v, delta)

    return call, jnp.asarray(delta_np)


def kernel(v):
    B, N, D = v.shape
    call, delta = _build(B, N, D)
    return call(v, delta)


# speedup vs baseline: 29.7295x; 1.0007x over previous
import functools
import numpy as np
import jax
import jax.numpy as jnp
from jax import lax
from jax.experimental import pallas as pl
from jax.experimental.pallas import tpu as pltpu
from jax.experimental.pallas import tpu_sc as plsc

_LAYER_IDX = 6
_NUM_LAYERS = 12
_DIM = 2048

_NSTRIPE = 16
_R = 128
_MAXOFF = 128
_SPAD = _R + 16
_S0 = _R + 64
_NBUF = 2


def _shift_offsets(v_len, d_v):
    i = np.arange(d_v, dtype=np.float64)
    e = (_LAYER_IDX * _DIM + i) / (_NUM_LAYERS * _DIM)
    return np.ceil(np.power(float(v_len), e)).astype(np.int64)


@functools.lru_cache(maxsize=None)
def _build(B, N, D):
    off = _shift_offsets(N, D)
    assert off.min() >= 1 and off.max() <= _MAXOFF
    delta_np = (_MAXOFF - off).astype(np.int32)
    CPW = D // _NSTRIPE
    NC16 = CPW // 16
    NBLK = N // _R
    T = (B * NBLK) // 2
    assert N % _R == 0 and D % _NSTRIPE == 0 and CPW % 16 == 0
    assert T % _NBUF == 0
    for s in range(_NSTRIPE):
        d = delta_np[s * CPW:(s + 1) * CPW]
        assert int(d.max()) - (int(d.min()) & ~7) <= _SPAD - _R

    mesh = plsc.VectorSubcoreMesh(core_axis_name="c", subcore_axis_name="s")

    @functools.partial(
        pl.kernel,
        out_type=jax.ShapeDtypeStruct((B, N, D), jnp.float32),
        mesh=mesh,
        compiler_params=pltpu.CompilerParams(needs_layout_passes=False),
        scratch_types=[
            pltpu.VMEM((CPW,), jnp.int32),
            pltpu.VMEM((_NBUF, _S0, CPW), jnp.float32),
            pltpu.VMEM((_NBUF, _R, CPW), jnp.float32),
            pltpu.SemaphoreType.DMA,
            pltpu.SemaphoreType.DMA,
            pltpu.SemaphoreType.DMA,
            pltpu.SemaphoreType.DMA,
        ],
    )
    def swd(v_hbm, delta_hbm, out_hbm, delta_v, src_v, dst_v,
            semi0, semi1, semo0, semo1):
        wid = lax.axis_index("s") * 2 + lax.axis_index("c")
        stripe = wid // 2
        half = wid % 2
        c0 = stripe * CPW
        t_base = half * T
        pltpu.sync_copy(delta_hbm.at[pl.ds(c0, CPW)], delta_v)
        iota = lax.iota(jnp.int32, 16)
        deltas = [delta_v[pl.ds(c * 16, 16)] for c in range(NC16)]
        cols = [c * 16 + iota for c in range(NC16)]
        dmin = deltas[0]
        for c in range(1, NC16):
            dmin = jnp.minimum(dmin, deltas[c])
        dmin8 = jnp.bitwise_and(lax.reduce_min(dmin, axes=(0,)),
                                jnp.int32(-8))
        semi = [semi0, semi1]
        semo = [semo0, semo1]

        def start_in(t, k, sem):
            tt = t_base + t
            b = tt // NBLK
            jblk = tt % NBLK
            j0 = jblk * _R
            sbuf = src_v.at[k]

            @pl.when(jblk == 0)
            def _():
                pltpu.async_copy(
                    v_hbm.at[b, pl.ds(N - _MAXOFF, _MAXOFF), pl.ds(c0, CPW)],
                    sbuf.at[pl.ds(0, _MAXOFF)], sem)
                pltpu.async_copy(
                    v_hbm.at[b, pl.ds(0, _S0 - _MAXOFF), pl.ds(c0, CPW)],
                    sbuf.at[pl.ds(_MAXOFF, _S0 - _MAXOFF)], sem)

            @pl.when(jblk != 0)
            def _():
                pltpu.async_copy(
                    v_hbm.at[b, pl.ds(pl.multiple_of(j0 - _MAXOFF + dmin8, 8),
                                      _SPAD),
                             pl.ds(c0, CPW)],
                    sbuf.at[pl.ds(0, _SPAD)], sem)

        def wait_in(t, k, sem):
            jblk = (t_base + t) % NBLK

            @pl.when(jblk == 0)
            def _():
                pltpu.make_async_copy(
                    v_hbm.at[0, pl.ds(0, _S0), pl.ds(c0, CPW)],
                    src_v.at[k], sem).wait()

            @pl.when(jblk != 0)
            def _():
                pltpu.make_async_copy(
                    v_hbm.at[0, pl.ds(0, _SPAD), pl.ds(c0, CPW)],
                    src_v.at[k, pl.ds(0, _SPAD)], sem).wait()

        def start_out(t, k, sem):
            tt = t_base + t
            b = tt // NBLK
            j0 = (tt % NBLK) * _R
            pltpu.async_copy(
                dst_v.at[k], out_hbm.at[b, pl.ds(j0, _R), pl.ds(c0, CPW)],
                sem)

        def wait_out(k, sem):
            pltpu.make_async_copy(
                dst_v.at[k], out_hbm.at[0, pl.ds(0, _R), pl.ds(c0, CPW)],
                sem).wait()

        def compute(t, k):
            jblk = (t_base + t) % NBLK
            dsub = jnp.where(jblk == 0, jnp.int32(0), dmin8)
            adj = [d - dsub for d in deltas]
            sbuf = src_v.at[k]
            dbuf = dst_v.at[k]

            @pl.loop(0, _R // 2, unroll=2)
            def pair(p):
                r = 2 * p
                for c in range(NC16):
                    ridx = adj[c] + r
                    lo = plsc.load_gather(sbuf, [ridx, cols[c]])
                    hi = plsc.load_gather(sbuf, [ridx + 1, cols[c]])
                    dbuf[r, pl.ds(c * 16, 16)] = jnp.minimum(lo, hi)
                    dbuf[r + 1, pl.ds(c * 16, 16)] = jnp.maximum(lo, hi)

        start_in(0, 0, semi[0])

        @pl.loop(0, T, step=_NBUF)
        def task_loop(t):
            for k in range(_NBUF):
                tk = t + k

                @pl.when(tk + 1 < T)
                def _():
                    start_in(tk + 1, (k + 1) % _NBUF, semi[(k + 1) % _NBUF])

                wait_in(tk, k, semi[k])

                @pl.when(tk >= _NBUF)
                def _():
                    wait_out(k, semo[k])

                compute(tk, k)
                start_out(tk, k, semo[k])

        for k in range(_NBUF):
            wait_out(k, semo[k])

    def call(v, delta):
        return swd(v, delta)

    return call, jnp.asarray(delta_np)


def kernel(v):
    B, N, D = v.shape
    call, delta = _build(B, N, D)
    return call(v, delta)


# speedup vs baseline: 49.9057x; 1.6787x over previous
import functools
import numpy as np
import jax
import jax.numpy as jnp
from jax import lax
from jax.experimental import pallas as pl
from jax.experimental.pallas import tpu as pltpu
from jax.experimental.pallas import tpu_sc as plsc

_LAYER_IDX = 6
_NUM_LAYERS = 12
_DIM = 2048

_NSTRIPE = 16
_R = 128
_MAXOFF = 128
_SPAD = _R + 16
_S0 = _R + 64
_NBUF = 2


def _shift_offsets(v_len, d_v):
    i = np.arange(d_v, dtype=np.float64)
    e = (_LAYER_IDX * _DIM + i) / (_NUM_LAYERS * _DIM)
    return np.ceil(np.power(float(v_len), e)).astype(np.int64)


@functools.lru_cache(maxsize=None)
def _build(B, N, D):
    off = _shift_offsets(N, D)
    assert off.min() >= 1 and off.max() <= _MAXOFF
    delta_np = (_MAXOFF - off).astype(np.int32)
    CPW = D // _NSTRIPE
    NC16 = CPW // 16
    NBLK = N // _R
    T = (B * NBLK) // 2
    assert N % _R == 0 and D % _NSTRIPE == 0 and CPW % 16 == 0
    assert T % _NBUF == 0
    for s in range(_NSTRIPE):
        d = delta_np[s * CPW:(s + 1) * CPW]
        assert int(d.max()) - (int(d.min()) & ~7) <= _SPAD - _R

    mesh = plsc.VectorSubcoreMesh(core_axis_name="c", subcore_axis_name="s")

    @functools.partial(
        pl.kernel,
        out_type=jax.ShapeDtypeStruct((B, N, D), jnp.float32),
        mesh=mesh,
        compiler_params=pltpu.CompilerParams(needs_layout_passes=False),
        scratch_types=[
            pltpu.VMEM((CPW,), jnp.int32),
            pltpu.VMEM((_NBUF, _S0, CPW), jnp.float32),
            pltpu.VMEM((_NBUF, _R, CPW), jnp.float32),
            pltpu.SemaphoreType.DMA,
            pltpu.SemaphoreType.DMA,
            pltpu.SemaphoreType.DMA,
            pltpu.SemaphoreType.DMA,
        ],
    )
    def swd(v_hbm, delta_hbm, out_hbm, delta_v, src_v, dst_v,
            semi0, semi1, semo0, semo1):
        wid = lax.axis_index("s") * 2 + lax.axis_index("c")
        stripe = wid // 2
        half = wid % 2
        c0 = stripe * CPW
        t_base = half * T
        pltpu.sync_copy(delta_hbm.at[pl.ds(c0, CPW)], delta_v)
        iota = lax.iota(jnp.int32, 16)
        deltas = [delta_v[pl.ds(c * 16, 16)] for c in range(NC16)]
        cols = [c * 16 + iota for c in range(NC16)]
        dmin = deltas[0]
        for c in range(1, NC16):
            dmin = jnp.minimum(dmin, deltas[c])
        dmin8 = jnp.bitwise_and(lax.reduce_min(dmin, axes=(0,)),
                                jnp.int32(-8))
        semi = [semi0, semi1]
        semo = [semo0, semo1]

        def start_in(t, k, sem):
            tt = t_base + t
            b = tt // NBLK
            jblk = tt % NBLK
            j0 = jblk * _R
            sbuf = src_v.at[k]

            @pl.when(jblk == 0)
            def _():
                pltpu.async_copy(
                    v_hbm.at[b, pl.ds(N - _MAXOFF, _MAXOFF), pl.ds(c0, CPW)],
                    sbuf.at[pl.ds(0, _MAXOFF)], sem)
                pltpu.async_copy(
                    v_hbm.at[b, pl.ds(0, _S0 - _MAXOFF), pl.ds(c0, CPW)],
                    sbuf.at[pl.ds(_MAXOFF, _S0 - _MAXOFF)], sem)

            @pl.when(jblk != 0)
            def _():
                pltpu.async_copy(
                    v_hbm.at[b, pl.ds(pl.multiple_of(j0 - _MAXOFF + dmin8, 8),
                                      _SPAD),
                             pl.ds(c0, CPW)],
                    sbuf.at[pl.ds(0, _SPAD)], sem)

        def wait_in(t, k, sem):
            jblk = (t_base + t) % NBLK

            @pl.when(jblk == 0)
            def _():
                pltpu.make_async_copy(
                    v_hbm.at[0, pl.ds(0, _S0), pl.ds(c0, CPW)],
                    src_v.at[k], sem).wait()

            @pl.when(jblk != 0)
            def _():
                pltpu.make_async_copy(
                    v_hbm.at[0, pl.ds(0, _SPAD), pl.ds(c0, CPW)],
                    src_v.at[k, pl.ds(0, _SPAD)], sem).wait()

        def start_out(t, k, sem):
            tt = t_base + t
            b = tt // NBLK
            j0 = (tt % NBLK) * _R
            pltpu.async_copy(
                dst_v.at[k], out_hbm.at[b, pl.ds(j0, _R), pl.ds(c0, CPW)],
                sem)

        def wait_out(k, sem):
            pltpu.make_async_copy(
                dst_v.at[k], out_hbm.at[0, pl.ds(0, _R), pl.ds(c0, CPW)],
                sem).wait()

        def compute(t, k):
            jblk = (t_base + t) % NBLK
            dsub = jnp.where(jblk == 0, jnp.int32(0), dmin8)
            base = [((deltas[c] - dsub) << 7) + cols[c] for c in range(NC16)]
            base2 = [b + CPW for b in base]
            zero16 = iota * 0
            sbuf = src_v.at[k]
            dbuf = dst_v.at[k]

            @pl.loop(0, _R // 2, unroll=2)
            def pair(p):
                r = 2 * p
                roff = r * CPW
                los = [plsc.load_gather(sbuf, [zero16, base[c] + roff])
                       for c in range(NC16)]
                his = [plsc.load_gather(sbuf, [zero16, base2[c] + roff])
                       for c in range(NC16)]
                for c in range(NC16):
                    dbuf[r, pl.ds(c * 16, 16)] = jnp.minimum(los[c], his[c])
                    dbuf[r + 1, pl.ds(c * 16, 16)] = jnp.maximum(los[c], his[c])

        start_in(0, 0, semi[0])

        @pl.loop(0, T, step=_NBUF)
        def task_loop(t):
            for k in range(_NBUF):
                tk = t + k

                @pl.when(tk + 1 < T)
                def _():
                    start_in(tk + 1, (k + 1) % _NBUF, semi[(k + 1) % _NBUF])

                wait_in(tk, k, semi[k])

                @pl.when(tk >= _NBUF)
                def _():
                    wait_out(k, semo[k])

                compute(tk, k)
                start_out(tk, k, semo[k])

        for k in range(_NBUF):
            wait_out(k, semo[k])

    def call(v, delta):
        return swd(v, delta)

    return call, jnp.asarray(delta_np)


def kernel(v):
    B, N, D = v.shape
    call, delta = _build(B, N, D)
    return call(v, delta)


# speedup vs baseline: 53.0951x; 1.0639x over previous
import functools
import numpy as np
import jax
import jax.numpy as jnp
from jax import lax
from jax.experimental import pallas as pl
from jax.experimental.pallas import tpu as pltpu
from jax.experimental.pallas import tpu_sc as plsc

_LAYER_IDX = 6
_NUM_LAYERS = 12
_DIM = 2048

_NSTRIPE = 16
_R = 128
_MAXOFF = 128
_SPAD = _R + 16
_S0 = _R + 64
_NBUF = 2


def _shift_offsets(v_len, d_v):
    i = np.arange(d_v, dtype=np.float64)
    e = (_LAYER_IDX * _DIM + i) / (_NUM_LAYERS * _DIM)
    return np.ceil(np.power(float(v_len), e)).astype(np.int64)


@functools.lru_cache(maxsize=None)
def _build(B, N, D):
    off = _shift_offsets(N, D)
    assert off.min() >= 1 and off.max() <= _MAXOFF
    delta_np = (_MAXOFF - off).astype(np.int32)
    CPW = D // _NSTRIPE
    NC16 = CPW // 16
    NBLK = N // _R
    T = (B * NBLK) // 2
    assert N % _R == 0 and D % _NSTRIPE == 0 and CPW % 16 == 0
    assert T % _NBUF == 0
    for s in range(_NSTRIPE):
        d = delta_np[s * CPW:(s + 1) * CPW]
        assert int(d.max()) - (int(d.min()) & ~7) <= _SPAD - _R

    mesh = plsc.VectorSubcoreMesh(core_axis_name="c", subcore_axis_name="s")

    @functools.partial(
        pl.kernel,
        out_type=jax.ShapeDtypeStruct((B, N, D), jnp.float32),
        mesh=mesh,
        compiler_params=pltpu.CompilerParams(needs_layout_passes=False),
        scratch_types=[
            pltpu.VMEM((CPW,), jnp.int32),
            pltpu.VMEM((_NBUF, _S0, CPW), jnp.float32),
            pltpu.VMEM((_NBUF, _R, CPW), jnp.float32),
            pltpu.SemaphoreType.DMA,
            pltpu.SemaphoreType.DMA,
            pltpu.SemaphoreType.DMA,
            pltpu.SemaphoreType.DMA,
        ],
    )
    def swd(v_hbm, delta_hbm, out_hbm, delta_v, src_v, dst_v,
            semi0, semi1, semo0, semo1):
        wid = lax.axis_index("s") * 2 + lax.axis_index("c")
        stripe = wid // 2
        half = wid % 2
        c0 = stripe * CPW
        t_base = half * T
        pltpu.sync_copy(delta_hbm.at[pl.ds(c0, CPW)], delta_v)
        iota = lax.iota(jnp.int32, 16)
        deltas = [delta_v[pl.ds(c * 16, 16)] for c in range(NC16)]
        cols = [c * 16 + iota for c in range(NC16)]
        dmin = deltas[0]
        for c in range(1, NC16):
            dmin = jnp.minimum(dmin, deltas[c])
        dmin8 = jnp.bitwise_and(lax.reduce_min(dmin, axes=(0,)),
                                jnp.int32(-8))
        semi = [semi0, semi1]
        semo = [semo0, semo1]

        def start_in(t, k, sem):
            tt = t_base + t
            b = tt // NBLK
            jblk = tt % NBLK
            j0 = jblk * _R
            sbuf = src_v.at[k]

            @pl.when(jblk == 0)
            def _():
                pltpu.async_copy(
                    v_hbm.at[b, pl.ds(N - _MAXOFF, _MAXOFF), pl.ds(c0, CPW)],
                    sbuf.at[pl.ds(0, _MAXOFF)], sem)
                pltpu.async_copy(
                    v_hbm.at[b, pl.ds(0, _S0 - _MAXOFF), pl.ds(c0, CPW)],
                    sbuf.at[pl.ds(_MAXOFF, _S0 - _MAXOFF)], sem)

            @pl.when(jblk != 0)
            def _():
                pltpu.async_copy(
                    v_hbm.at[b, pl.ds(pl.multiple_of(j0 - _MAXOFF + dmin8, 8),
                                      _SPAD),
                             pl.ds(c0, CPW)],
                    sbuf.at[pl.ds(0, _SPAD)], sem)

        def wait_in(t, k, sem):
            jblk = (t_base + t) % NBLK

            @pl.when(jblk == 0)
            def _():
                pltpu.make_async_copy(
                    v_hbm.at[0, pl.ds(0, _S0), pl.ds(c0, CPW)],
                    src_v.at[k], sem).wait()

            @pl.when(jblk != 0)
            def _():
                pltpu.make_async_copy(
                    v_hbm.at[0, pl.ds(0, _SPAD), pl.ds(c0, CPW)],
                    src_v.at[k, pl.ds(0, _SPAD)], sem).wait()

        def start_out(t, k, sem):
            tt = t_base + t
            b = tt // NBLK
            j0 = (tt % NBLK) * _R
            pltpu.async_copy(
                dst_v.at[k], out_hbm.at[b, pl.ds(j0, _R), pl.ds(c0, CPW)],
                sem)

        def wait_out(k, sem):
            pltpu.make_async_copy(
                dst_v.at[k], out_hbm.at[0, pl.ds(0, _R), pl.ds(c0, CPW)],
                sem).wait()

        def compute(t, k):
            jblk = (t_base + t) % NBLK
            dsub = jnp.where(jblk == 0, jnp.int32(0), dmin8)
            base = [((deltas[c] - dsub) << 7) + cols[c] for c in range(NC16)]
            base2 = [b + CPW for b in base]
            zero16 = iota * 0
            sbuf = src_v.at[k]
            dbuf = dst_v.at[k]

            @plsc.parallel_loop(0, _R // 2, unroll=2)
            def pair(p):
                r = 2 * p
                roff = r * CPW
                los = [plsc.load_gather(sbuf, [zero16, base[c] + roff])
                       for c in range(NC16)]
                his = [plsc.load_gather(sbuf, [zero16, base2[c] + roff])
                       for c in range(NC16)]
                for c in range(NC16):
                    dbuf[r, pl.ds(c * 16, 16)] = jnp.minimum(los[c], his[c])
                    dbuf[r + 1, pl.ds(c * 16, 16)] = jnp.maximum(los[c], his[c])

        start_in(0, 0, semi[0])

        @pl.loop(0, T, step=_NBUF)
        def task_loop(t):
            for k in range(_NBUF):
                tk = t + k

                @pl.when(tk + 1 < T)
                def _():
                    start_in(tk + 1, (k + 1) % _NBUF, semi[(k + 1) % _NBUF])

                wait_in(tk, k, semi[k])

                @pl.when(tk >= _NBUF)
                def _():
                    wait_out(k, semo[k])

                compute(tk, k)
                start_out(tk, k, semo[k])

        for k in range(_NBUF):
            wait_out(k, semo[k])

    def call(v, delta):
        return swd(v, delta)

    return call, jnp.asarray(delta_np)


def kernel(v):
    B, N, D = v.shape
    call, delta = _build(B, N, D)
    return call(v, delta)
